# baseline (speedup 1.0000x reference)
import jax
import jax.numpy as jnp
from jax.experimental import pallas as pl
from jax.experimental.pallas import tpu as pltpu

B, H, S, D = 8, 16, 4096, 128
Q = 16
BLK = 1024


def _body(pos_ref, kval_ref, vval_ref, kc_ref, vc_ref, ko_ref, vo_ref):
    j = pl.program_id(1)
    base = j * BLK
    ko_ref[...] = kc_ref[...]
    vo_ref[...] = vc_ref[...]
    for q in range(Q):
        p = pos_ref[q]
        off = p - base

        @pl.when(jnp.logical_and(p >= base, p < base + BLK))
        def _():
            ko_ref[0, pl.ds(off, 1), :] = kval_ref[0, pl.ds(q, 1), :]
            vo_ref[0, pl.ds(off, 1), :] = vval_ref[0, pl.ds(q, 1), :]


def kernel(input_pos, k_val, v_val, k_cache, v_cache):
    kc = k_cache.reshape(B * H, S, D)
    vc = v_cache.reshape(B * H, S, D)
    kv = k_val.reshape(B * H, Q, D)
    vv = v_val.reshape(B * H, Q, D)
    grid = (B * H, S // BLK)
    cache_spec = pl.BlockSpec((1, BLK, D), lambda i, j: (i, j, 0))
    val_spec = pl.BlockSpec((1, Q, D), lambda i, j: (i, 0, 0))
    ko, vo = pl.pallas_call(
        _body,
        grid=grid,
        in_specs=[
            pl.BlockSpec(memory_space=pltpu.SMEM),
            val_spec,
            val_spec,
            cache_spec,
            cache_spec,
        ],
        out_specs=[cache_spec, cache_spec],
        out_shape=[
            jax.ShapeDtypeStruct((B * H, S, D), jnp.float32),
            jax.ShapeDtypeStruct((B * H, S, D), jnp.float32),
        ],
        compiler_params=pltpu.CompilerParams(
            dimension_semantics=("arbitrary", "arbitrary"),
        ),
    )(input_pos, kv, vv, kc, vc)
    return ko.reshape(B, H, S, D), vo.reshape(B, H, S, D)
